# baseline (speedup 1.0000x reference)
import jax
import jax.numpy as jnp
from jax.experimental import pallas as pl
from jax.experimental.pallas import tpu as pltpu

_N, _L, _T = 16384, 200, 16
_W = _L * _T
_C = _W // 128
_CHUNKS = 8
_BN = 1024


def _decode_kernel(e_ref, tags_ref, out_ref):
    bn = e_ref.shape[0]
    key = e_ref[...]
    col = jax.lax.broadcasted_iota(jnp.int32, (bn, _W), 1)
    lfull = col // 16

    lane = jax.lax.broadcasted_iota(jnp.int32, (bn, 128), 1)
    lane_t = lane % 16
    lane_g = lane // 16
    tags = tags_ref[...]
    ninf = jnp.float32(-jnp.inf)

    out_keys = jnp.zeros((bn, 128), jnp.float32)
    for i in range(_CHUNKS):
        parts = [key[:, j * 128:(j + 1) * 128] for j in range(_C)]
        while len(parts) > 1:
            nxt = [jnp.maximum(parts[p], parts[p + 1])
                   for p in range(0, len(parts) - 1, 2)]
            if len(parts) % 2:
                nxt.append(parts[-1])
            parts = nxt
        v = parts[0]
        for sh in (64, 32, 16):
            v = jnp.maximum(v, pltpu.roll(v, sh, 1))
        out_keys = jnp.where(lane_g == i, v, out_keys)
        tk = jnp.max(jnp.where(lane_t == tags[:, i:i + 1], v, ninf),
                     axis=1, keepdims=True)
        tb = jax.lax.bitcast_convert_type(tk, jnp.int32)
        tlow = tb & jnp.int32(255)
        lsel = jnp.where(tb >= 0, jnp.int32(255) - tlow, tlow)
        key = jnp.where(lfull == lsel, ninf, key)

    kb = jax.lax.bitcast_convert_type(out_keys, jnp.int32)
    kd = (kb & jnp.int32(-256)) | jnp.int32(128)
    out_ref[...] = jax.lax.bitcast_convert_type(kd, jnp.float32)


@jax.jit
def kernel(emissions, tags):
    n, l, t = emissions.shape
    e = emissions.reshape(n, _W)
    b = jax.lax.bitcast_convert_type(e, jnp.int32)
    col = jax.lax.broadcasted_iota(jnp.int32, (n, _W), 1)
    lf = col // 16
    low8 = jnp.where(b >= 0, jnp.int32(255) - lf, lf)
    e = jax.lax.bitcast_convert_type((b & jnp.int32(-256)) | low8, jnp.float32)
    tg = tags.astype(jnp.int32)
    out = pl.pallas_call(
        _decode_kernel,
        grid=(n // _BN,),
        in_specs=[
            pl.BlockSpec((_BN, _W), lambda i: (i, 0)),
            pl.BlockSpec((_BN, _CHUNKS), lambda i: (i, 0)),
        ],
        out_specs=pl.BlockSpec((_BN, 128), lambda i: (i, 0)),
        out_shape=jax.ShapeDtypeStruct((n, 128), jnp.float32),
        compiler_params=pltpu.CompilerParams(
            dimension_semantics=("parallel",),
        ),
    )(e, tg)
    return out.reshape(n, _CHUNKS, t)


# speedup vs baseline: 1.7191x; 1.7191x over previous
import jax
import jax.numpy as jnp
from jax.experimental import pallas as pl
from jax.experimental.pallas import tpu as pltpu

_N, _L, _T = 16384, 200, 16
_W = _L * _T
_C = _W // 128
_CHUNKS = 8
_BN = 1024


def _decode_kernel(e_ref, tags_ref, out_ref):
    bn = e_ref.shape[0]
    x = e_ref[...]
    b = jax.lax.bitcast_convert_type(x, jnp.int32)
    col = jax.lax.broadcasted_iota(jnp.int32, (bn, _W), 1)
    lfull = col // 16
    low8 = jnp.where(b >= 0, jnp.int32(255) - lfull, lfull)
    key = jax.lax.bitcast_convert_type((b & jnp.int32(-256)) | low8,
                                       jnp.float32)

    lane = jax.lax.broadcasted_iota(jnp.int32, (bn, 128), 1)
    lane_t = lane % 16
    lane_g = lane // 16
    tags = tags_ref[...]
    ninf = jnp.float32(-jnp.inf)

    out_keys = jnp.zeros((bn, 128), jnp.float32)
    for i in range(_CHUNKS):
        parts = [key[:, j * 128:(j + 1) * 128] for j in range(_C)]
        while len(parts) > 1:
            nxt = [jnp.maximum(parts[p], parts[p + 1])
                   for p in range(0, len(parts) - 1, 2)]
            if len(parts) % 2:
                nxt.append(parts[-1])
            parts = nxt
        v = parts[0]
        for sh in (64, 32, 16):
            v = jnp.maximum(v, pltpu.roll(v, sh, 1))
        out_keys = jnp.where(lane_g == i, v, out_keys)
        tk = jnp.max(jnp.where(lane_t == tags[:, i:i + 1], v, ninf),
                     axis=1, keepdims=True)
        tb = jax.lax.bitcast_convert_type(tk, jnp.int32)
        tlow = tb & jnp.int32(255)
        lsel = jnp.where(tb >= 0, jnp.int32(255) - tlow, tlow)
        key = jnp.where(lfull == lsel, ninf, key)

    kb = jax.lax.bitcast_convert_type(out_keys, jnp.int32)
    kd = (kb & jnp.int32(-256)) | jnp.int32(128)
    out_ref[...] = jax.lax.bitcast_convert_type(kd, jnp.float32)


@jax.jit
def kernel(emissions, tags):
    n, l, t = emissions.shape
    e = emissions.reshape(n, _W)
    tg = tags.astype(jnp.int32)
    out = pl.pallas_call(
        _decode_kernel,
        grid=(n // _BN,),
        in_specs=[
            pl.BlockSpec((_BN, _W), lambda i: (i, 0)),
            pl.BlockSpec((_BN, _CHUNKS), lambda i: (i, 0)),
        ],
        out_specs=pl.BlockSpec((_BN, 128), lambda i: (i, 0)),
        out_shape=jax.ShapeDtypeStruct((n, 128), jnp.float32),
        compiler_params=pltpu.CompilerParams(
            dimension_semantics=("arbitrary",),
        ),
    )(e, tg)
    return out.reshape(n, _CHUNKS, t)


# speedup vs baseline: 1.7193x; 1.0001x over previous
import jax
import jax.numpy as jnp
from jax.experimental import pallas as pl
from jax.experimental.pallas import tpu as pltpu

_N, _L, _T = 16384, 200, 16
_W = _L * _T
_C = _W // 128
_CHUNKS = 8
_BN = 1024


def _decode_kernel(e_ref, tags_ref, out_ref):
    bn = e_ref.shape[0]
    x = e_ref[...]
    b = jax.lax.bitcast_convert_type(x, jnp.int32)
    col = jax.lax.broadcasted_iota(jnp.int32, (bn, _W), 1)
    lfull = col // 16
    low8 = jnp.where(b >= 0, jnp.int32(255) - lfull, lfull)
    key = jax.lax.bitcast_convert_type((b & jnp.int32(-256)) | low8,
                                       jnp.float32)

    lane = jax.lax.broadcasted_iota(jnp.int32, (bn, 128), 1)
    lane_t = lane % 16
    lane_g = lane // 16
    tags = tags_ref[...]
    ninf = jnp.float32(-jnp.inf)

    out_keys = jnp.zeros((bn, 128), jnp.float32)
    for i in range(_CHUNKS):
        parts = [key[:, j * 128:(j + 1) * 128] for j in range(_C)]
        while len(parts) > 1:
            nxt = [jnp.maximum(parts[p], parts[p + 1])
                   for p in range(0, len(parts) - 1, 2)]
            if len(parts) % 2:
                nxt.append(parts[-1])
            parts = nxt
        v = parts[0]
        for sh in (64, 32, 16):
            v = jnp.maximum(v, pltpu.roll(v, sh, 1))
        out_keys = jnp.where(lane_g == i, v, out_keys)
        if i == _CHUNKS - 1:
            break
        tk = jnp.max(jnp.where(lane_t == tags[:, i:i + 1], v, ninf),
                     axis=1, keepdims=True)
        tb = jax.lax.bitcast_convert_type(tk, jnp.int32)
        tlow = tb & jnp.int32(255)
        lsel = jnp.where(tb >= 0, jnp.int32(255) - tlow, tlow)
        key = jnp.where(lfull == lsel, ninf, key)

    kb = jax.lax.bitcast_convert_type(out_keys, jnp.int32)
    kd = (kb & jnp.int32(-256)) | jnp.int32(128)
    out_ref[...] = jax.lax.bitcast_convert_type(kd, jnp.float32)


@jax.jit
def kernel(emissions, tags):
    n, l, t = emissions.shape
    e = emissions.reshape(n, _W)
    tg = tags.astype(jnp.int32)
    out = pl.pallas_call(
        _decode_kernel,
        grid=(n // _BN,),
        in_specs=[
            pl.BlockSpec((_BN, _W), lambda i: (i, 0)),
            pl.BlockSpec((_BN, _CHUNKS), lambda i: (i, 0)),
        ],
        out_specs=pl.BlockSpec((_BN, 128), lambda i: (i, 0)),
        out_shape=jax.ShapeDtypeStruct((n, 128), jnp.float32),
        compiler_params=pltpu.CompilerParams(
            dimension_semantics=("parallel",),
        ),
    )(e, tg)
    return out.reshape(n, _CHUNKS, t)


# speedup vs baseline: 1.7222x; 1.0017x over previous
import jax
import jax.numpy as jnp
from jax.experimental import pallas as pl
from jax.experimental.pallas import tpu as pltpu

_N, _L, _T = 16384, 200, 16
_W = _L * _T
_C = _W // 128
_CHUNKS = 8
_BN = 1024


def _decode_kernel(e_ref, tags_ref, out_ref, key_ref):
    bn = e_ref.shape[0]
    x = e_ref[...]
    b = jax.lax.bitcast_convert_type(x, jnp.int32)
    col = jax.lax.broadcasted_iota(jnp.int32, (bn, _W), 1)
    lfull = col // 16
    low8 = jnp.where(b >= 0, jnp.int32(255) - lfull, lfull)
    key_ref[...] = jax.lax.bitcast_convert_type((b & jnp.int32(-256)) | low8,
                                                jnp.float32)

    lane = jax.lax.broadcasted_iota(jnp.int32, (bn, 128), 1)
    lane_t = lane % 16
    lane_g = lane // 16
    tags = tags_ref[...]
    ninf = jnp.float32(-jnp.inf)

    out_keys = jnp.zeros((bn, 128), jnp.float32)
    for i in range(_CHUNKS):
        parts = [key_ref[:, j * 128:(j + 1) * 128] for j in range(_C)]
        while len(parts) > 1:
            nxt = [jnp.maximum(parts[p], parts[p + 1])
                   for p in range(0, len(parts) - 1, 2)]
            if len(parts) % 2:
                nxt.append(parts[-1])
            parts = nxt
        v = parts[0]
        for sh in (64, 32, 16):
            v = jnp.maximum(v, pltpu.roll(v, sh, 1))
        out_keys = jnp.where(lane_g == i, v, out_keys)
        if i == _CHUNKS - 1:
            break
        tk = jnp.max(jnp.where(lane_t == tags[:, i:i + 1], v, ninf),
                     axis=1, keepdims=True)
        tb = jax.lax.bitcast_convert_type(tk, jnp.int32)
        tlow = tb & jnp.int32(255)
        lsel = jnp.where(tb >= 0, jnp.int32(255) - tlow, tlow)
        for j in range(_C):
            sl = slice(j * 128, (j + 1) * 128)
            eq = lfull[:, sl] == lsel
            key_ref[:, sl] = jnp.where(eq, ninf, key_ref[:, sl])

    kb = jax.lax.bitcast_convert_type(out_keys, jnp.int32)
    kd = (kb & jnp.int32(-256)) | jnp.int32(128)
    out_ref[...] = jax.lax.bitcast_convert_type(kd, jnp.float32)


@jax.jit
def kernel(emissions, tags):
    n, l, t = emissions.shape
    e = emissions.reshape(n, _W)
    tg = tags.astype(jnp.int32)
    out = pl.pallas_call(
        _decode_kernel,
        grid=(n // _BN,),
        in_specs=[
            pl.BlockSpec((_BN, _W), lambda i: (i, 0)),
            pl.BlockSpec((_BN, _CHUNKS), lambda i: (i, 0)),
        ],
        out_specs=pl.BlockSpec((_BN, 128), lambda i: (i, 0)),
        out_shape=jax.ShapeDtypeStruct((n, 128), jnp.float32),
        scratch_shapes=[pltpu.VMEM((_BN, _W), jnp.float32)],
        compiler_params=pltpu.CompilerParams(
            dimension_semantics=("parallel",),
        ),
    )(e, tg)
    return out.reshape(n, _CHUNKS, t)
